# Initial kernel scaffold; baseline (speedup 1.0000x reference)
import jax
import jax.numpy as jnp
from jax.experimental import pallas as pl


def kernel(adj, features, W_base, W_mean, W_logstd, noise):
    raise NotImplementedError("write your pallas kernel here")


# speedup vs baseline: 1.2453x; 1.2453x over previous
import jax
import jax.numpy as jnp
from jax import lax
from jax.experimental import pallas as pl


def _p_body(f_ref, wb_ref, wcat_ref, p_ref):
    wc = jnp.dot(wb_ref[...], wcat_ref[...], preferred_element_type=jnp.float32)
    p_ref[...] = jnp.dot(f_ref[...], wc, preferred_element_type=jnp.float32)


def _g_body(a_ref, p_ref, g_ref):
    g_ref[...] = jnp.dot(a_ref[...], p_ref[...],
                         preferred_element_type=jnp.float32)


def _z_body(a_ref, g_ref, noise_ref, z_ref, d_emb: int):
    m = jnp.dot(a_ref[...], g_ref[...], preferred_element_type=jnp.float32)
    m = jnp.maximum(m, 0.0)
    mean = m[:, :d_emb]
    logstd = m[:, d_emb:]
    z_ref[...] = noise_ref[...] * jnp.exp(logstd) + mean


def _out_body(zi_ref, z_ref, o_ref):
    o_ref[...] = lax.dot_general(
        zi_ref[...], z_ref[...], (((1,), (1,)), ((), ())),
        preferred_element_type=jnp.float32)


def kernel(adj, features, W_base, W_mean, W_logstd, noise):
    n, d_in = features.shape
    d_hid = W_base.shape[1]
    d_emb = W_mean.shape[1]
    d2 = 2 * d_emb

    w_cat = jnp.concatenate([W_mean, W_logstd], axis=1)

    p = pl.pallas_call(
        _p_body,
        out_shape=jax.ShapeDtypeStruct((n, d2), jnp.float32),
    )(features, W_base, w_cat)

    bm = 400
    grid = (n // bm,)

    g = pl.pallas_call(
        _g_body,
        grid=grid,
        in_specs=[
            pl.BlockSpec((bm, n), lambda i: (i, 0)),
            pl.BlockSpec((n, d2), lambda i: (0, 0)),
        ],
        out_specs=pl.BlockSpec((bm, d2), lambda i: (i, 0)),
        out_shape=jax.ShapeDtypeStruct((n, d2), jnp.float32),
    )(adj, p)

    z = pl.pallas_call(
        lambda a, gg, nz, zz: _z_body(a, gg, nz, zz, d_emb),
        grid=grid,
        in_specs=[
            pl.BlockSpec((bm, n), lambda i: (i, 0)),
            pl.BlockSpec((n, d2), lambda i: (0, 0)),
            pl.BlockSpec((bm, d_emb), lambda i: (i, 0)),
        ],
        out_specs=pl.BlockSpec((bm, d_emb), lambda i: (i, 0)),
        out_shape=jax.ShapeDtypeStruct((n, d_emb), jnp.float32),
    )(adj, g, noise)

    out = pl.pallas_call(
        _out_body,
        grid=grid,
        in_specs=[
            pl.BlockSpec((bm, d_emb), lambda i: (i, 0)),
            pl.BlockSpec((n, d_emb), lambda i: (0, 0)),
        ],
        out_specs=pl.BlockSpec((bm, n), lambda i: (i, 0)),
        out_shape=jax.ShapeDtypeStruct((n, n), jnp.float32),
    )(z, z)

    return out


# speedup vs baseline: 1.3057x; 1.0485x over previous
import jax
import jax.numpy as jnp
from jax import lax
from jax.experimental import pallas as pl
from jax.experimental.pallas import tpu as pltpu

_BM = 200


def _body(adj_ref, f_ref, wb_ref, wcat_ref, noise_ref, o_ref,
          p_ref, g_ref, z_ref, *, nb, d_emb):
    i = pl.program_id(0)

    @pl.when(i == 0)
    def _phase_p():
        wc = jnp.dot(wb_ref[...], wcat_ref[...],
                     preferred_element_type=jnp.float32)
        p_ref[...] = jnp.dot(f_ref[...], wc,
                             preferred_element_type=jnp.float32)

    @pl.when((i >= 1) & (i <= nb))
    def _phase_g():
        r = (i - 1) * _BM
        g_ref[pl.ds(r, _BM), :] = jnp.dot(
            adj_ref[...], p_ref[...], preferred_element_type=jnp.float32)

    @pl.when((i >= nb + 1) & (i <= 2 * nb))
    def _phase_z():
        r = (i - 1 - nb) * _BM
        m = jnp.maximum(jnp.dot(adj_ref[...], g_ref[...],
                                preferred_element_type=jnp.float32), 0.0)
        mean = m[:, :d_emb]
        logstd = m[:, d_emb:]
        z_ref[pl.ds(r, _BM), :] = (
            noise_ref[pl.ds(r, _BM), :] * jnp.exp(logstd) + mean)

    @pl.when(i > 2 * nb)
    def _phase_out():
        r = (i - 1 - 2 * nb) * _BM
        zi = z_ref[pl.ds(r, _BM), :]
        o_ref[...] = lax.dot_general(
            zi, z_ref[...], (((1,), (1,)), ((), ())),
            preferred_element_type=jnp.float32)


def kernel(adj, features, W_base, W_mean, W_logstd, noise):
    n, d_in = features.shape
    d_hid = W_base.shape[1]
    d_emb = W_mean.shape[1]
    d2 = 2 * d_emb
    nb = n // _BM

    w_cat = jnp.concatenate([W_mean, W_logstd], axis=1)

    def adj_map(i):
        return (jnp.where(i <= nb, jnp.maximum(i - 1, 0),
                          jnp.where(i <= 2 * nb, i - 1 - nb, nb - 1)), 0)

    def out_map(i):
        return (jnp.where(i > 2 * nb, i - 1 - 2 * nb, 0), 0)

    import functools
    body = functools.partial(_body, nb=nb, d_emb=d_emb)

    out = pl.pallas_call(
        body,
        grid=(3 * nb + 1,),
        in_specs=[
            pl.BlockSpec((_BM, n), adj_map),
            pl.BlockSpec((n, d_in), lambda i: (0, 0)),
            pl.BlockSpec((d_in, d_hid), lambda i: (0, 0)),
            pl.BlockSpec((d_hid, d2), lambda i: (0, 0)),
            pl.BlockSpec((n, d_emb), lambda i: (0, 0)),
        ],
        out_specs=pl.BlockSpec((_BM, n), out_map),
        out_shape=jax.ShapeDtypeStruct((n, n), jnp.float32),
        scratch_shapes=[
            pltpu.VMEM((n, d2), jnp.float32),
            pltpu.VMEM((n, d2), jnp.float32),
            pltpu.VMEM((n, d_emb), jnp.float32),
        ],
    )(adj, features, W_base, w_cat, noise)

    return out
